# baseline (speedup 1.0000x reference)
import functools

import jax
import jax.numpy as jnp
from jax.experimental import pallas as pl
from jax.experimental.pallas import tpu as pltpu


def _chamfer_body(p1_ref, p2t_ref, out_ref, d2_ref, *, n1, n2):
    b = pl.program_id(0)
    i = pl.program_id(1)
    ni = pl.num_programs(1)

    x = p1_ref[0]
    y = p2t_ref[0]
    tm = x.shape[0]

    x2 = jnp.sum(x * x, axis=1, keepdims=True)
    x2_hi = x2.astype(jnp.bfloat16)
    x2_lo = (x2 - x2_hi.astype(jnp.float32)).astype(jnp.bfloat16)
    ones_x = jnp.ones((tm, 2), jnp.bfloat16)
    lhs = jnp.concatenate(
        [(-2.0 * x).astype(jnp.bfloat16), x2_hi, x2_lo, ones_x], axis=1
    )

    y2 = jnp.sum(y * y, axis=0, keepdims=True)
    y2_hi = y2.astype(jnp.bfloat16)
    y2_lo = (y2 - y2_hi.astype(jnp.float32)).astype(jnp.bfloat16)
    ones_y = jnp.ones((2, y.shape[1]), jnp.bfloat16)
    rhs = jnp.concatenate(
        [y.astype(jnp.bfloat16), ones_y, y2_hi, y2_lo], axis=0
    )

    w = jax.lax.dot_general(
        lhs, rhs, (((1,), (0,)), ((), ())),
        preferred_element_type=jnp.float32,
    ).astype(jnp.bfloat16)

    m1 = jnp.maximum(jnp.min(w, axis=1).astype(jnp.float32), 0.0)
    m2 = jnp.min(w, axis=0, keepdims=True)

    @pl.when(jnp.logical_and(b == 0, i == 0))
    def _init_out():
        out_ref[...] = jnp.zeros((1, 1), jnp.float32)

    @pl.when(i == 0)
    def _init_d2():
        d2_ref[...] = m2

    @pl.when(i > 0)
    def _acc_d2():
        d2_ref[...] = jnp.minimum(d2_ref[...], m2)

    out_ref[...] += jnp.reshape(jnp.sum(m1), (1, 1)) * (1.0 / n1)

    @pl.when(i == ni - 1)
    def _finish_batch():
        d2 = jnp.maximum(d2_ref[...].astype(jnp.float32), 0.0)
        out_ref[...] += jnp.reshape(jnp.sum(d2), (1, 1)) * (1.0 / n2)


def kernel(points1, points2):
    B, N1, C = points1.shape
    _, N2, _ = points2.shape
    p2t = jnp.transpose(points2, (0, 2, 1))

    TM = 1024
    grid = (B, N1 // TM)

    out = pl.pallas_call(
        functools.partial(_chamfer_body, n1=N1, n2=N2),
        grid=grid,
        in_specs=[
            pl.BlockSpec((1, TM, C), lambda b, i: (b, i, 0)),
            pl.BlockSpec((1, C, N2), lambda b, i: (b, 0, 0)),
        ],
        out_specs=pl.BlockSpec((1, 1), lambda b, i: (0, 0)),
        out_shape=jax.ShapeDtypeStruct((1, 1), jnp.float32),
        scratch_shapes=[pltpu.VMEM((1, N2), jnp.bfloat16)],
    )(points1, p2t)
    return out[0, 0]


# speedup vs baseline: 1.1182x; 1.1182x over previous
import functools

import jax
import jax.numpy as jnp
from jax.experimental import pallas as pl
from jax.experimental.pallas import tpu as pltpu


def _chamfer_body(p1_ref, p2t_ref, out_ref, d2_ref, *, n1, n2):
    b = pl.program_id(0)
    i = pl.program_id(1)
    ni = pl.num_programs(1)

    x = p1_ref[0]
    y = p2t_ref[0]
    tm = x.shape[0]

    x2 = jnp.sum(x * x, axis=1, keepdims=True)
    x2_hi = x2.astype(jnp.bfloat16)
    x2_lo = (x2 - x2_hi.astype(jnp.float32)).astype(jnp.bfloat16)
    ones_x = jnp.ones((tm, 2), jnp.bfloat16)
    lhs = jnp.concatenate(
        [(-2.0 * x).astype(jnp.bfloat16), x2_hi, x2_lo, ones_x], axis=1
    )

    y2 = jnp.sum(y * y, axis=0, keepdims=True)
    y2_hi = y2.astype(jnp.bfloat16)
    y2_lo = (y2 - y2_hi.astype(jnp.float32)).astype(jnp.bfloat16)
    ones_y = jnp.ones((2, y.shape[1]), jnp.bfloat16)
    rhs = jnp.concatenate(
        [y.astype(jnp.bfloat16), ones_y, y2_hi, y2_lo], axis=0
    )

    w = jax.lax.dot_general(
        lhs, rhs, (((1,), (0,)), ((), ())),
        preferred_element_type=jnp.float32,
    ).astype(jnp.bfloat16)

    m1 = jnp.maximum(jnp.min(w, axis=1).astype(jnp.float32), 0.0)
    m2 = jnp.min(w, axis=0, keepdims=True)

    @pl.when(jnp.logical_and(b == 0, i == 0))
    def _init_out():
        out_ref[...] = jnp.zeros((1, 1), jnp.float32)

    @pl.when(i == 0)
    def _init_d2():
        d2_ref[...] = m2

    @pl.when(i > 0)
    def _acc_d2():
        d2_ref[...] = jnp.minimum(d2_ref[...], m2)

    out_ref[...] += jnp.reshape(jnp.sum(m1), (1, 1)) * (1.0 / n1)

    @pl.when(i == ni - 1)
    def _finish_batch():
        d2 = jnp.maximum(d2_ref[...].astype(jnp.float32), 0.0)
        out_ref[...] += jnp.reshape(jnp.sum(d2), (1, 1)) * (1.0 / n2)


def kernel(points1, points2):
    B, N1, C = points1.shape
    _, N2, _ = points2.shape
    p2t = jnp.transpose(points2, (0, 2, 1))

    TM = 4096
    grid = (B, N1 // TM)

    out = pl.pallas_call(
        functools.partial(_chamfer_body, n1=N1, n2=N2),
        grid=grid,
        in_specs=[
            pl.BlockSpec((1, TM, C), lambda b, i: (b, i, 0)),
            pl.BlockSpec((1, C, N2), lambda b, i: (b, 0, 0)),
        ],
        out_specs=pl.BlockSpec((1, 1), lambda b, i: (0, 0)),
        out_shape=jax.ShapeDtypeStruct((1, 1), jnp.float32),
        scratch_shapes=[pltpu.VMEM((1, N2), jnp.bfloat16)],
    )(points1, p2t)
    return out[0, 0]


# speedup vs baseline: 1.1345x; 1.0145x over previous
import functools

import jax
import jax.numpy as jnp
from jax.experimental import pallas as pl


def _chamfer_body(p1_ref, p2t_ref, out_ref, *, n1, n2):
    b = pl.program_id(0)

    x = p1_ref[0]
    y = p2t_ref[0]

    x2 = jnp.sum(x * x, axis=1, keepdims=True)
    x2_hi = x2.astype(jnp.bfloat16)
    x2_lo = (x2 - x2_hi.astype(jnp.float32)).astype(jnp.bfloat16)
    ones_x = jnp.ones((x.shape[0], 2), jnp.bfloat16)
    lhs = jnp.concatenate(
        [(-2.0 * x).astype(jnp.bfloat16), x2_hi, x2_lo, ones_x], axis=1
    )

    y2 = jnp.sum(y * y, axis=0, keepdims=True)
    y2_hi = y2.astype(jnp.bfloat16)
    y2_lo = (y2 - y2_hi.astype(jnp.float32)).astype(jnp.bfloat16)
    ones_y = jnp.ones((2, y.shape[1]), jnp.bfloat16)
    rhs = jnp.concatenate(
        [y.astype(jnp.bfloat16), ones_y, y2_hi, y2_lo], axis=0
    )

    w = jax.lax.dot_general(
        lhs, rhs, (((1,), (0,)), ((), ())),
        preferred_element_type=jnp.float32,
    ).astype(jnp.bfloat16)

    m1 = jnp.maximum(jnp.min(w, axis=1).astype(jnp.float32), 0.0)
    m2 = jnp.maximum(
        jnp.min(w, axis=0, keepdims=True).astype(jnp.float32), 0.0)

    cost = jnp.sum(m1) * (1.0 / n1) + jnp.sum(m2) * (1.0 / n2)

    @pl.when(b == 0)
    def _init_out():
        out_ref[...] = jnp.zeros((1, 1), jnp.float32)

    out_ref[...] += jnp.reshape(cost, (1, 1))


def kernel(points1, points2):
    B, N1, C = points1.shape
    _, N2, _ = points2.shape
    p2t = jnp.transpose(points2, (0, 2, 1))

    out = pl.pallas_call(
        functools.partial(_chamfer_body, n1=N1, n2=N2),
        grid=(B,),
        in_specs=[
            pl.BlockSpec((1, N1, C), lambda b: (b, 0, 0)),
            pl.BlockSpec((1, C, N2), lambda b: (b, 0, 0)),
        ],
        out_specs=pl.BlockSpec((1, 1), lambda b: (0, 0)),
        out_shape=jax.ShapeDtypeStruct((1, 1), jnp.float32),
    )(points1, p2t)
    return out[0, 0]


# speedup vs baseline: 1.1409x; 1.0057x over previous
import functools

import jax
import jax.numpy as jnp
from jax.experimental import pallas as pl


def _batch_cost(x, y, n1, n2):
    x2 = jnp.sum(x * x, axis=1, keepdims=True)
    x2_hi = x2.astype(jnp.bfloat16)
    x2_lo = (x2 - x2_hi.astype(jnp.float32)).astype(jnp.bfloat16)
    ones_x = jnp.ones((x.shape[0], 2), jnp.bfloat16)
    lhs = jnp.concatenate(
        [(-2.0 * x).astype(jnp.bfloat16), x2_hi, x2_lo, ones_x], axis=1
    )

    y2 = jnp.sum(y * y, axis=0, keepdims=True)
    y2_hi = y2.astype(jnp.bfloat16)
    y2_lo = (y2 - y2_hi.astype(jnp.float32)).astype(jnp.bfloat16)
    ones_y = jnp.ones((2, y.shape[1]), jnp.bfloat16)
    rhs = jnp.concatenate(
        [y.astype(jnp.bfloat16), ones_y, y2_hi, y2_lo], axis=0
    )

    w = jax.lax.dot_general(
        lhs, rhs, (((1,), (0,)), ((), ())),
        preferred_element_type=jnp.float32,
    ).astype(jnp.bfloat16)

    m1 = jnp.maximum(jnp.min(w, axis=1).astype(jnp.float32), 0.0)
    m2 = jnp.maximum(
        jnp.min(w, axis=0, keepdims=True).astype(jnp.float32), 0.0)

    return jnp.sum(m1) * (1.0 / n1) + jnp.sum(m2) * (1.0 / n2)


def _chamfer_body(p1_ref, p2t_ref, out_ref, *, n1, n2, bps):
    b = pl.program_id(0)

    cost = _batch_cost(p1_ref[0], p2t_ref[0], n1, n2)
    for k in range(1, bps):
        cost += _batch_cost(p1_ref[k], p2t_ref[k], n1, n2)

    @pl.when(b == 0)
    def _init_out():
        out_ref[...] = jnp.zeros((1, 1), jnp.float32)

    out_ref[...] += jnp.reshape(cost, (1, 1))


def kernel(points1, points2):
    B, N1, C = points1.shape
    _, N2, _ = points2.shape
    p2t = jnp.transpose(points2, (0, 2, 1))

    BPS = 2
    out = pl.pallas_call(
        functools.partial(_chamfer_body, n1=N1, n2=N2, bps=BPS),
        grid=(B // BPS,),
        in_specs=[
            pl.BlockSpec((BPS, N1, C), lambda b: (b, 0, 0)),
            pl.BlockSpec((BPS, C, N2), lambda b: (b, 0, 0)),
        ],
        out_specs=pl.BlockSpec((1, 1), lambda b: (0, 0)),
        out_shape=jax.ShapeDtypeStruct((1, 1), jnp.float32),
    )(points1, p2t)
    return out[0, 0]
